# baseline (speedup 1.0000x reference)
import functools

import jax
import jax.numpy as jnp
from jax import lax
from jax.experimental import pallas as pl
from jax.experimental.pallas import tpu as pltpu
from jax.experimental.pallas import tpu_sc as plsc

VOCAB = 100000
BIGRAM_VOCAB = 1000000
BIGRAM_DIM = 32
MODEL_DIM = 768

NC = 2
NS = 16
NW = NC * NS
B_TOTAL = 4 * 4096
CHUNK = B_TOTAL // NW
N_VREG = CHUNK // 16
GATHER_W = 128
N_GATHER = CHUNK // GATHER_W

_sc_mesh = plsc.VectorSubcoreMesh(
    core_axis_name="c", subcore_axis_name="s", num_cores=NC, num_subcores=NS
)


@functools.partial(
    pl.kernel,
    out_type=jax.ShapeDtypeStruct((2, B_TOTAL, BIGRAM_DIM), jnp.float32),
    mesh=_sc_mesh,
    compiler_params=pltpu.CompilerParams(use_tc_tiling_on_sc=False),
    scratch_types=[
        pltpu.VMEM((CHUNK,), jnp.int32),
        pltpu.VMEM((CHUNK,), jnp.int32),
        pltpu.VMEM((N_GATHER, GATHER_W), jnp.int32),
        pltpu.VMEM((N_GATHER, GATHER_W), jnp.int32),
        pltpu.VMEM((CHUNK, BIGRAM_DIM), jnp.float32),
        pltpu.VMEM((CHUNK, BIGRAM_DIM), jnp.float32),
        pltpu.SemaphoreType.DMA,
    ],
)
def _sc_gather(x_hbm, prev_hbm, e1_hbm, e2_hbm, out_hbm,
               x_v, prev_v, idx1_v, idx2_v, rows1_v, rows2_v, sem):
    wid = lax.axis_index("s") * NC + lax.axis_index("c")
    base = wid * CHUNK
    pltpu.sync_copy(x_hbm.at[pl.ds(base, CHUNK)], x_v)
    pltpu.sync_copy(prev_hbm.at[pl.ds(base, CHUNK)], prev_v)
    for i in range(N_VREG):
        xa = x_v[pl.ds(i * 16, 16)]
        pa = prev_v[pl.ds(i * 16, 16)]
        h1 = (pa * 1024 + xa) % BIGRAM_VOCAB
        h2 = (pa + xa * 31) % BIGRAM_VOCAB
        j, c = divmod(i, GATHER_W // 16)
        idx1_v[j, pl.ds(c * 16, 16)] = h1
        idx2_v[j, pl.ds(c * 16, 16)] = h2
    copies = []
    for j in range(N_GATHER):
        copies.append(pltpu.async_copy(
            e1_hbm.at[idx1_v.at[j]], rows1_v.at[pl.ds(j * GATHER_W, GATHER_W)], sem))
        copies.append(pltpu.async_copy(
            e2_hbm.at[idx2_v.at[j]], rows2_v.at[pl.ds(j * GATHER_W, GATHER_W)], sem))
    for cp in copies:
        cp.wait()
    pltpu.sync_copy(rows1_v, out_hbm.at[0, pl.ds(base, CHUNK)])
    pltpu.sync_copy(rows2_v, out_hbm.at[1, pl.ds(base, CHUNK)])


M_BLK = 2048


def _tc_matmul_body(gate_ref, e_ref, w_ref, o_ref):
    g = jax.nn.sigmoid(gate_ref[0])
    w1 = w_ref[:, :BIGRAM_DIM] * g
    w2 = w_ref[:, BIGRAM_DIM:] * (1.0 - g)
    acc = lax.dot_general(e_ref[0], w1, (((1,), (1,)), ((), ())),
                          preferred_element_type=jnp.float32)
    acc += lax.dot_general(e_ref[1], w2, (((1,), (1,)), ((), ())),
                           preferred_element_type=jnp.float32)
    o_ref[...] = acc


def _tc_matmul(gate_flat, e_both, proj_w):
    return pl.pallas_call(
        _tc_matmul_body,
        grid=(B_TOTAL // M_BLK,),
        in_specs=[
            pl.BlockSpec(memory_space=pltpu.SMEM),
            pl.BlockSpec((2, M_BLK, BIGRAM_DIM), lambda i: (0, i, 0)),
            pl.BlockSpec((MODEL_DIM, 2 * BIGRAM_DIM), lambda i: (0, 0)),
        ],
        out_specs=pl.BlockSpec((M_BLK, MODEL_DIM), lambda i: (i, 0)),
        out_shape=jax.ShapeDtypeStruct((B_TOTAL, MODEL_DIM), jnp.float32),
    )(gate_flat, e_both, proj_w)


def kernel(x, embed1, embed2, proj_w, gate):
    batch, seqlen = x.shape
    x_flat = x.reshape(-1)
    prev_flat = jnp.pad(x[:, :-1], ((0, 0), (1, 0)), constant_values=0).reshape(-1)
    e_both = _sc_gather(x_flat, prev_flat, embed1, embed2)
    y = _tc_matmul(gate.reshape(-1), e_both, proj_w)
    return y.reshape(batch, seqlen, MODEL_DIM)
